# Initial kernel scaffold; baseline (speedup 1.0000x reference)
import functools

import jax
import jax.numpy as jnp
from jax import lax
from jax.experimental import pallas as pl
from jax.experimental.pallas import tpu as pltpu
from jax.experimental.pallas import tpu_sc as plsc

VOCAB = 100000
NUM_SPECIAL = 3
DIM = 4096
LANES = 16
NC, NS = 2, 16
NW = NC * NS
N_TOK = 1024 * 20
PER_W = N_TOK // NW
CHUNK = 8
NBUF = 2
NCHUNK = PER_W // CHUNK
TOKBUF = PER_W + LANES - CHUNK


def _body(tok_hbm, table_hbm, embed_hbm, out_hbm,
          tok_v, emb_v, buf0, buf1, gsem0, gsem1, ssem0, ssem1):
    wid = lax.axis_index("s") * NC + lax.axis_index("c")
    base = wid * PER_W

    pltpu.sync_copy(tok_hbm.at[pl.ds(base, PER_W)], tok_v.at[pl.ds(0, PER_W)])
    pltpu.sync_copy(embed_hbm, emb_v)

    lane = lax.iota(jnp.int32, 16)
    tail = tok_v[pl.ds(PER_W - CHUNK, LANES)]
    tok_v[pl.ds(PER_W - CHUNK, LANES)] = jnp.where(lane < CHUNK, tail, 0)

    bufs = (buf0, buf1)
    gsems = (gsem0, gsem1)
    ssems = (ssem0, ssem1)

    def idx_ref(g):
        return tok_v.at[pl.ds(g * CHUNK, CHUNK)]

    def start_gather(g, b):
        pltpu.async_copy(table_hbm.at[idx_ref(g)], bufs[b], gsems[b])

    def wait_gather(g, b):
        pltpu.make_async_copy(table_hbm.at[idx_ref(g)], bufs[b], gsems[b]).wait()

    def out_ref(g):
        return out_hbm.at[pl.ds(base + g * CHUNK, CHUNK)]

    def start_store(g, b):
        pltpu.async_copy(bufs[b], out_ref(g), ssems[b])

    def wait_store(g, b):
        pltpu.make_async_copy(bufs[b], out_ref(g), ssems[b]).wait()

    def fixup(g, b):
        tokw = tok_v[pl.ds(g * CHUNK, LANES)]
        spec = (tokw >= VOCAB) & (lane < CHUNK)
        any_spec = jnp.max(spec.astype(jnp.int32))

        @pl.when(any_spec > 0)
        def _():
            eidx = jnp.clip(tokw - VOCAB, 0, NUM_SPECIAL - 1)

            def col(c, carry):
                cvec = jnp.full((LANES,), 0, jnp.int32) + c
                vals = plsc.load_gather(emb_v, [eidx, cvec], mask=spec)
                plsc.store_scatter(bufs[b], [lane, cvec], vals, mask=spec)
                return carry

            lax.fori_loop(0, DIM, col, 0)

    for b in range(NBUF):
        start_gather(b, b)

    def step(it, carry):
        for b in range(NBUF):
            g = it * NBUF + b
            wait_gather(g, b)
            fixup(g, b)
            start_store(g, b)

            @pl.when(g + NBUF < NCHUNK)
            def _():
                wait_store(g, b)
                start_gather(g + NBUF, b)

        return carry

    lax.fori_loop(0, NCHUNK // NBUF, step, 0)

    for b in range(NBUF):
        wait_store(NCHUNK - NBUF + b, b)


@jax.jit
def _run(tokens_flat, table, embed16):
    mesh = plsc.VectorSubcoreMesh(
        core_axis_name="c", subcore_axis_name="s",
        num_cores=NC, num_subcores=NS)
    f = pl.kernel(
        _body,
        out_type=jax.ShapeDtypeStruct((N_TOK, DIM), jnp.float32),
        mesh=mesh,
        scratch_types=[
            pltpu.VMEM((TOKBUF,), jnp.int32),
            pltpu.VMEM((NUM_SPECIAL, DIM), jnp.float32),
            pltpu.VMEM((CHUNK, DIM), jnp.float32),
            pltpu.VMEM((CHUNK, DIM), jnp.float32),
            pltpu.SemaphoreType.DMA,
            pltpu.SemaphoreType.DMA,
            pltpu.SemaphoreType.DMA,
            pltpu.SemaphoreType.DMA,
        ],
    )
    return f(tokens_flat, table, embed16)


def kernel(tokens, table, embed):
    embed16 = embed.astype(jnp.float16).astype(jnp.float32)
    out = _run(tokens.reshape(-1), table, embed16)
    return out.reshape(tokens.shape[0], tokens.shape[1], DIM)


# speedup vs baseline: 1.3149x; 1.3149x over previous
import functools

import jax
import jax.numpy as jnp
from jax import lax
from jax.experimental import pallas as pl
from jax.experimental.pallas import tpu as pltpu
from jax.experimental.pallas import tpu_sc as plsc

VOCAB = 100000
NUM_SPECIAL = 3
DIM = 4096
LANES = 16
NC, NS = 2, 16
NW = NC * NS
N_TOK = 1024 * 20
PER_W = N_TOK // NW
CHUNK = 8
NBUF = 2
NCHUNK = PER_W // CHUNK
TOKBUF = PER_W + LANES - CHUNK


def _body(tok_hbm, table_hbm, embed_hbm, out_hbm,
          tok_v, emb_v, buf0, buf1, gsem0, gsem1, ssem0, ssem1):
    wid = lax.axis_index("s") * NC + lax.axis_index("c")
    base = wid * PER_W

    pltpu.sync_copy(tok_hbm.at[pl.ds(base, PER_W)], tok_v.at[pl.ds(0, PER_W)])
    pltpu.sync_copy(embed_hbm, emb_v)

    lane = lax.iota(jnp.int32, 16)
    tail = tok_v[pl.ds(PER_W - CHUNK, LANES)]
    tok_v[pl.ds(PER_W - CHUNK, LANES)] = jnp.where(lane < CHUNK, tail, 0)

    bufs = (buf0, buf1)
    gsems = (gsem0, gsem1)
    ssems = (ssem0, ssem1)

    def idx_ref(g):
        return tok_v.at[pl.ds(g * CHUNK, CHUNK)]

    def start_gather(g, b):
        pltpu.async_copy(table_hbm.at[idx_ref(g)], bufs[b], gsems[b])

    def wait_gather(g, b):
        pltpu.make_async_copy(table_hbm.at[idx_ref(g)], bufs[b], gsems[b]).wait()

    def out_ref(g):
        return out_hbm.at[pl.ds(base + g * CHUNK, CHUNK)]

    def start_store(g, b):
        pltpu.async_copy(bufs[b], out_ref(g), ssems[b])

    def wait_store(g, b):
        pltpu.make_async_copy(bufs[b], out_ref(g), ssems[b]).wait()

    def fixup(g, b):
        tokw = tok_v[pl.ds(g * CHUNK, LANES)]
        spec = (tokw >= VOCAB) & (lane < CHUNK)
        any_spec = jnp.max(spec.astype(jnp.int32))

        @pl.when(any_spec > 0)
        def _():
            eidx = jnp.clip(tokw - VOCAB, 0, NUM_SPECIAL - 1)

            def col(c, carry):
                cvec = jnp.full((LANES,), 0, jnp.int32) + c
                vals = plsc.load_gather(emb_v, [eidx, cvec], mask=spec)
                plsc.store_scatter(bufs[b], [lane, cvec], vals, mask=spec)
                return carry

            lax.fori_loop(0, DIM, col, 0)

    for b in range(NBUF):
        start_gather(b, b)

    def step(it, carry):
        for b in range(NBUF):
            g = it * NBUF + b
            wait_gather(g, b)
            fixup(g, b)
            start_store(g, b)

            @pl.when(g + NBUF < NCHUNK)
            def _():
                wait_store(g, b)
                start_gather(g + NBUF, b)

        return carry

    lax.fori_loop(0, NCHUNK // NBUF, step, 0)

    for b in range(NBUF):
        wait_store(NCHUNK - NBUF + b, b)


@jax.jit
def _run(tokens_flat, table, embed16):
    mesh = plsc.VectorSubcoreMesh(
        core_axis_name="c", subcore_axis_name="s",
        num_cores=NC, num_subcores=NS)
    f = pl.kernel(
        _body,
        out_type=jax.ShapeDtypeStruct((N_TOK, DIM), jnp.float32),
        mesh=mesh,
        scratch_types=[
            pltpu.VMEM((TOKBUF,), jnp.int32),
            pltpu.VMEM((NUM_SPECIAL, DIM), jnp.float32),
            pltpu.VMEM((CHUNK, DIM), jnp.float32),
            pltpu.VMEM((CHUNK, DIM), jnp.float32),
            pltpu.SemaphoreType.DMA,
            pltpu.SemaphoreType.DMA,
            pltpu.SemaphoreType.DMA,
            pltpu.SemaphoreType.DMA,
        ],
        compiler_params=pltpu.CompilerParams(needs_layout_passes=False),
    )
    return f(tokens_flat, table, embed16)


def kernel(tokens, table, embed):
    embed16 = embed.astype(jnp.float16).astype(jnp.float32)
    out = _run(tokens.reshape(-1), table, embed16)
    return out.reshape(tokens.shape[0], tokens.shape[1], DIM)


# speedup vs baseline: 1.3253x; 1.0079x over previous
import functools

import jax
import jax.numpy as jnp
from jax import lax
from jax.experimental import pallas as pl
from jax.experimental.pallas import tpu as pltpu
from jax.experimental.pallas import tpu_sc as plsc

VOCAB = 100000
NUM_SPECIAL = 3
DIM = 4096
LANES = 16
NC, NS = 2, 16
NW = NC * NS
N_TOK = 1024 * 20
PER_W = N_TOK // NW
CHUNK = 8
NBUF = 3
NCHUNK = PER_W // CHUNK
TOKBUF = PER_W + LANES - CHUNK


def _body(tok_hbm, table_hbm, embed_hbm, out_hbm,
          tok_v, emb_v, buf0, buf1, buf2,
          gsem0, gsem1, gsem2, ssem0, ssem1, ssem2):
    wid = lax.axis_index("s") * NC + lax.axis_index("c")
    base = wid * PER_W

    pltpu.sync_copy(tok_hbm.at[pl.ds(base, PER_W)], tok_v.at[pl.ds(0, PER_W)])
    pltpu.sync_copy(embed_hbm, emb_v)

    lane = lax.iota(jnp.int32, 16)
    tail = tok_v[pl.ds(PER_W - CHUNK, LANES)]
    tok_v[pl.ds(PER_W - CHUNK, LANES)] = jnp.where(lane < CHUNK, tail, 0)

    bufs = (buf0, buf1, buf2)
    gsems = (gsem0, gsem1, gsem2)
    ssems = (ssem0, ssem1, ssem2)

    def idx_ref(g):
        return tok_v.at[pl.ds(g * CHUNK, CHUNK)]

    def start_gather(g, b):
        pltpu.async_copy(table_hbm.at[idx_ref(g)], bufs[b], gsems[b])

    def wait_gather(g, b):
        pltpu.make_async_copy(table_hbm.at[idx_ref(g)], bufs[b], gsems[b]).wait()

    def out_ref(g):
        return out_hbm.at[pl.ds(base + g * CHUNK, CHUNK)]

    def start_store(g, b):
        pltpu.async_copy(bufs[b], out_ref(g), ssems[b])

    def wait_store(g, b):
        pltpu.make_async_copy(bufs[b], out_ref(g), ssems[b]).wait()

    def fixup(g, b):
        tokw = tok_v[pl.ds(g * CHUNK, LANES)]
        spec = (tokw >= VOCAB) & (lane < CHUNK)
        any_spec = jnp.max(spec.astype(jnp.int32))

        @pl.when(any_spec > 0)
        def _():
            eidx = jnp.clip(tokw - VOCAB, 0, NUM_SPECIAL - 1)

            def col(c, carry):
                cvec = jnp.full((LANES,), 0, jnp.int32) + c
                vals = plsc.load_gather(emb_v, [eidx, cvec], mask=spec)
                plsc.store_scatter(bufs[b], [lane, cvec], vals, mask=spec)
                return carry

            lax.fori_loop(0, DIM, col, 0)

    for b in range(NBUF):
        start_gather(b, b)

    def step(it, carry):
        for b in range(NBUF):
            g = it * NBUF + b
            wait_gather(g, b)
            fixup(g, b)
            start_store(g, b)

            @pl.when(g + NBUF < NCHUNK)
            def _():
                wait_store(g, b)
                start_gather(g + NBUF, b)

        return carry

    n_full = NCHUNK // NBUF
    lax.fori_loop(0, n_full, step, 0)

    for g in range(n_full * NBUF, NCHUNK):
        b = g % NBUF
        wait_gather(g, b)
        fixup(g, b)
        start_store(g, b)

    for g in range(NCHUNK - NBUF, NCHUNK):
        wait_store(g, g % NBUF)


@jax.jit
def _run(tokens_flat, table, embed16):
    mesh = plsc.VectorSubcoreMesh(
        core_axis_name="c", subcore_axis_name="s",
        num_cores=NC, num_subcores=NS)
    f = pl.kernel(
        _body,
        out_type=jax.ShapeDtypeStruct((N_TOK, DIM), jnp.float32),
        mesh=mesh,
        scratch_types=[
            pltpu.VMEM((TOKBUF,), jnp.int32),
            pltpu.VMEM((NUM_SPECIAL, DIM), jnp.float32),
            pltpu.VMEM((CHUNK, DIM), jnp.float32),
            pltpu.VMEM((CHUNK, DIM), jnp.float32),
            pltpu.VMEM((CHUNK, DIM), jnp.float32),
            pltpu.SemaphoreType.DMA,
            pltpu.SemaphoreType.DMA,
            pltpu.SemaphoreType.DMA,
            pltpu.SemaphoreType.DMA,
            pltpu.SemaphoreType.DMA,
            pltpu.SemaphoreType.DMA,
        ],
        compiler_params=pltpu.CompilerParams(needs_layout_passes=False),
    )
    return f(tokens_flat, table, embed16)


def kernel(tokens, table, embed):
    embed16 = embed.astype(jnp.float16).astype(jnp.float32)
    out = _run(tokens.reshape(-1), table, embed16)
    return out.reshape(tokens.shape[0], tokens.shape[1], DIM)


# speedup vs baseline: 1.9664x; 1.4838x over previous
import functools

import jax
import jax.numpy as jnp
from jax import lax
from jax.experimental import pallas as pl
from jax.experimental.pallas import tpu as pltpu
from jax.experimental.pallas import tpu_sc as plsc

VOCAB = 100000
NUM_SPECIAL = 3
DIM = 4096
LANES = 16
NC, NS = 2, 16
NW = NC * NS
BATCH = 1024
SEQ = 20
SEQ_PAD = 24
NB_W = BATCH // NW
PER_W = NB_W * SEQ
CHUNKS = ((0, 8), (8, 8), (16, 4))
NBUF = 3
BUFROWS = 8
TOKPAD = NB_W * SEQ_PAD + LANES


def _body(tok_hbm, table_hbm, embed_hbm, out_hbm,
          tok_v, tok_p, emb_v, buf0, buf1, buf2,
          gsem0, gsem1, gsem2, ssem0, ssem1, ssem2):
    wid = lax.axis_index("s") * NC + lax.axis_index("c")
    base = wid * PER_W
    batch0 = wid * NB_W

    pltpu.sync_copy(tok_hbm.at[pl.ds(base, PER_W)], tok_v)
    pltpu.sync_copy(embed_hbm, emb_v)

    lane = lax.iota(jnp.int32, LANES)

    def zero_step(i, carry):
        tok_p[pl.ds(i * LANES, LANES)] = jnp.zeros((LANES,), jnp.int32)
        return carry

    lax.fori_loop(0, TOKPAD // LANES, zero_step, 0)

    def pack_step(i, carry):
        t = i * LANES + lane
        dst = (t // SEQ) * SEQ_PAD + (t % SEQ)
        plsc.store_scatter(tok_p, [dst], tok_v[pl.ds(i * LANES, LANES)])
        return carry

    lax.fori_loop(0, PER_W // LANES, pack_step, 0)

    bufs = (buf0, buf1, buf2)
    gsems = (gsem0, gsem1, gsem2)
    ssems = (ssem0, ssem1, ssem2)

    def idx_ref(bi, ci):
        off, ln = CHUNKS[ci]
        return tok_p.at[pl.ds(bi * SEQ_PAD + off, ln)]

    def gather_descr(bi, ci, b):
        return (table_hbm.at[idx_ref(bi, ci)], bufs[b], gsems[b])

    def store_descr(bi, ci, b):
        off, ln = CHUNKS[ci]
        return (bufs[b], out_hbm.at[batch0 + bi, pl.ds(off, ln)], ssems[b])

    def start_gather(bi, ci, b):
        src, dst, sem = gather_descr(bi, ci, b)
        pltpu.async_copy(src, dst, sem)

    def wait_gather(bi, ci, b):
        src, dst, sem = gather_descr(bi, ci, b)
        pltpu.make_async_copy(src, dst, sem).wait()

    def start_store(bi, ci, b):
        src, dst, sem = store_descr(bi, ci, b)
        pltpu.async_copy(src, dst, sem)

    def wait_store(bi, ci, b):
        src, dst, sem = store_descr(bi, ci, b)
        pltpu.make_async_copy(src, dst, sem).wait()

    def fixup(bi, ci, b):
        off, ln = CHUNKS[ci]
        tokw = tok_p[pl.ds(bi * SEQ_PAD + off, LANES)]
        spec = (tokw >= VOCAB) & (lane < ln)
        any_spec = jnp.max(spec.astype(jnp.int32))

        @pl.when(any_spec > 0)
        def _():
            eidx = jnp.clip(tokw - VOCAB, 0, NUM_SPECIAL - 1)

            def col(c, carry):
                cvec = jnp.full((LANES,), 0, jnp.int32) + c
                vals = plsc.load_gather(emb_v, [eidx, cvec], mask=spec)
                plsc.store_scatter(bufs[b], [lane, cvec], vals, mask=spec)
                return carry

            lax.fori_loop(0, DIM, col, 0)

    for k in range(NBUF):
        start_gather(k // 3, k % 3, k % NBUF)

    NCH = 3 * NB_W

    def step(it, carry):
        for ci in range(3):
            k = it * 3 + ci
            bi = it
            b = ci
            wait_gather(bi, ci, b)
            fixup(bi, ci, b)
            start_store(bi, ci, b)

            @pl.when(k + NBUF < NCH)
            def _():
                nk = k + NBUF
                wait_store(bi, ci, b)
                start_gather(it + 1, ci, b)

        return carry

    lax.fori_loop(0, NB_W, step, 0)

    for ci in range(3):
        wait_store(NB_W - 1, ci, ci)


@jax.jit
def _run(tokens_flat, table, embed16):
    mesh = plsc.VectorSubcoreMesh(
        core_axis_name="c", subcore_axis_name="s",
        num_cores=NC, num_subcores=NS)
    f = pl.kernel(
        _body,
        out_type=jax.ShapeDtypeStruct((BATCH, SEQ, DIM), jnp.float32),
        mesh=mesh,
        scratch_types=[
            pltpu.VMEM((PER_W,), jnp.int32),
            pltpu.VMEM((TOKPAD,), jnp.int32),
            pltpu.VMEM((NUM_SPECIAL, DIM), jnp.float32),
            pltpu.VMEM((CHUNKS[0][1], DIM), jnp.float32),
            pltpu.VMEM((CHUNKS[1][1], DIM), jnp.float32),
            pltpu.VMEM((CHUNKS[2][1], DIM), jnp.float32),
            pltpu.SemaphoreType.DMA,
            pltpu.SemaphoreType.DMA,
            pltpu.SemaphoreType.DMA,
            pltpu.SemaphoreType.DMA,
            pltpu.SemaphoreType.DMA,
            pltpu.SemaphoreType.DMA,
        ],
        compiler_params=pltpu.CompilerParams(needs_layout_passes=False),
    )
    return f(tokens_flat, table, embed16)


def kernel(tokens, table, embed):
    embed16 = embed.astype(jnp.float16).astype(jnp.float32)
    return _run(tokens.reshape(-1), table, embed16)
